# baseline (speedup 1.0000x reference)
import functools

import jax
import jax.numpy as jnp
from jax.experimental import pallas as pl
from jax.experimental.pallas import tpu as pltpu

V = 100000
D = 512
B = 16
BEAM = 4
STEPS = 6
PAD, BOS, EOS = 0, 1, 2
LEN_PEN = 0.6
NEG_INF = -1e9

VC = 10000
NC = V // VC
LANES = 128
NROWS = 64
NSUB = LANES // NROWS
SUB = VC // NSUB


def _chunk_kernel(iota_ref, hT_ref, emb_ref, m_ref, s_ref, v_ref, i_ref,
                  acc_m, acc_s, acc_v, acc_i):
    j = pl.program_id(0)
    row = iota_ref[...]
    lane = jax.lax.broadcasted_iota(jnp.int32, (1, LANES), 1)
    sub_off = jnp.where(lane >= NROWS, SUB, 0)

    tiles = [
        jax.lax.dot_general(
            emb_ref[s * SUB:(s + 1) * SUB, :], hT_ref[...],
            dimension_numbers=(((1,), (0,)), ((), ())),
            preferred_element_type=jnp.float32)
        for s in range(NSUB)
    ]
    logits = jnp.concatenate(tiles, axis=1)

    m = jnp.max(logits, axis=0)[None, :]
    s = jnp.sum(jnp.exp(logits - m), axis=0)[None, :]

    vals = logits
    cv, ci = [], []
    mt = m[0]
    for t in range(BEAM):
        if t:
            mt = jnp.max(vals, axis=0)
        ki = jnp.where(vals == mt[None, :], row, SUB)
        at = jnp.min(ki, axis=0)
        cv.append(mt)
        ci.append(j * VC + at + sub_off[0])
        if t + 1 < BEAM:
            vals = jnp.where(ki == at[None, :], NEG_INF, vals)
    nv = jnp.stack(cv, axis=0)
    ni = jnp.stack(ci, axis=0)

    @pl.when(j == 0)
    def _init():
        acc_m[...] = m
        acc_s[...] = s
        acc_v[...] = nv
        acc_i[...] = ni

    @pl.when(j > 0)
    def _accum():
        m_old = acc_m[...]
        m_new = jnp.maximum(m_old, m)
        acc_s[...] = (acc_s[...] * jnp.exp(m_old - m_new)
                      + s * jnp.exp(m - m_new))
        acc_m[...] = m_new
        sv = jnp.concatenate([acc_v[...], nv], axis=0)
        si = jnp.concatenate([acc_i[...], ni], axis=0)
        for t in range(BEAM):
            mv = jnp.max(sv, axis=0)
            hit = sv == mv[None, :]
            cidx = jnp.min(jnp.where(hit, si, V), axis=0)
            acc_v[t, :] = mv
            acc_i[t, :] = cidx
            sv = jnp.where(hit & (si == cidx[None, :]), NEG_INF, sv)

    @pl.when(j == NC - 1)
    def _emit():
        m_ref[...] = acc_m[...]
        s_ref[...] = acc_s[...]
        v_ref[...] = acc_v[...]
        i_ref[...] = acc_i[...]


def _stream_pass(hT, emb_table):
    iota = jnp.broadcast_to(
        jnp.arange(SUB, dtype=jnp.int32)[:, None], (SUB, LANES))
    return pl.pallas_call(
        _chunk_kernel,
        grid=(NC,),
        in_specs=[
            pl.BlockSpec((SUB, LANES), lambda j: (0, 0)),
            pl.BlockSpec((D, NROWS), lambda j: (0, 0)),
            pl.BlockSpec((VC, D), lambda j: (j, 0)),
        ],
        out_specs=[
            pl.BlockSpec((1, LANES), lambda j: (0, 0)),
            pl.BlockSpec((1, LANES), lambda j: (0, 0)),
            pl.BlockSpec((BEAM, LANES), lambda j: (0, 0)),
            pl.BlockSpec((BEAM, LANES), lambda j: (0, 0)),
        ],
        out_shape=[
            jax.ShapeDtypeStruct((1, LANES), jnp.float32),
            jax.ShapeDtypeStruct((1, LANES), jnp.float32),
            jax.ShapeDtypeStruct((BEAM, LANES), jnp.float32),
            jax.ShapeDtypeStruct((BEAM, LANES), jnp.int32),
        ],
        scratch_shapes=[
            pltpu.VMEM((1, LANES), jnp.float32),
            pltpu.VMEM((1, LANES), jnp.float32),
            pltpu.VMEM((BEAM, LANES), jnp.float32),
            pltpu.VMEM((BEAM, LANES), jnp.int32),
        ],
    )(iota, hT, emb_table)


def _vocab_topk(hT, emb_table):
    m_c, s_c, v_c, i_c = _stream_pass(hT, emb_table)
    m2 = m_c.reshape(NSUB, NROWS)
    s2 = s_c.reshape(NSUB, NROWS)
    mx = jnp.max(m2, axis=0)
    lse = mx + jnp.log(jnp.sum(s2 * jnp.exp(m2 - mx[None, :]), axis=0))
    vals = v_c.reshape(BEAM, NSUB, NROWS).transpose(1, 0, 2) \
        .reshape(NSUB * BEAM, NROWS).T
    idx = i_c.reshape(BEAM, NSUB, NROWS).transpose(1, 0, 2) \
        .reshape(NSUB * BEAM, NROWS).T
    top_v, pos = jax.lax.top_k(vals, BEAM)
    top_i = jnp.take_along_axis(idx, pos, axis=1)
    return top_v, top_i, lse


def kernel(decoder_input_ids, encoder_hidden_states, encoder_input_mask,
           emb_table, pos_emb, W_dec, W_enc):
    mask = encoder_input_mask
    enc_ctx = (encoder_hidden_states * mask[:, :, None]).sum(axis=1) / \
        jnp.maximum(mask.sum(axis=1, keepdims=True), 1.0)

    tok0 = decoder_input_ids[:, 0]
    h = jnp.take(emb_table, tok0, axis=0) + pos_emb[0][None, :]
    h = jnp.tanh(h @ W_dec + enc_ctx @ W_enc)
    h = jnp.tile(h, (NROWS // B, 1))
    top_v, top_i, lse = _vocab_topk(h.T, emb_table)
    top_v, top_i, lse = top_v[:B], top_i[:B], lse[:B]
    scores = (top_v - lse[:, None]).reshape(-1, 1)
    prefixes = jnp.concatenate(
        [jnp.repeat(decoder_input_ids, BEAM, axis=0),
         top_i.reshape(-1, 1).astype(jnp.int32)], axis=1)
    ctx_rep = jnp.repeat(enc_ctx, BEAM, axis=0)
    ctx_enc = ctx_rep @ W_enc
    pad_profile = jnp.zeros((B * BEAM,), dtype=jnp.int32)

    for i in range(1, STEPS + 1):
        tok = prefixes[:, -1]
        h = jnp.take(emb_table, tok, axis=0) + pos_emb[i][None, :]
        h = jnp.tanh(h @ W_dec + ctx_enc)
        top_v, top_i, lse = _vocab_topk(h.T, emb_table)
        lp_cand = top_v - lse[:, None]
        finished = pad_profile > 0
        pad_lp = jnp.where(jnp.arange(BEAM) == 0, 0.0, NEG_INF)
        lp_cand = jnp.where(finished[:, None], pad_lp[None, :], lp_cand)
        tok_cand = jnp.where(finished[:, None], PAD, top_i)

        total = scores + lp_cand
        length = prefixes.shape[1]
        penalty = ((5.0 + length) / 6.0) ** LEN_PEN
        cand = (total / penalty).reshape(B, BEAM * BEAM)
        _, tpos = jax.lax.top_k(cand, BEAM)
        beam_idx = tpos // BEAM
        token = jnp.take_along_axis(
            tok_cand.reshape(B, BEAM * BEAM), tpos, axis=1).astype(jnp.int32)
        new_scores = jnp.take_along_axis(
            total.reshape(B, BEAM * BEAM), tpos, axis=1).reshape(-1, 1)
        flat_beam = (beam_idx + jnp.arange(B)[:, None] * BEAM).reshape(-1)
        prefixes = jnp.concatenate(
            [prefixes[flat_beam], token.reshape(-1, 1)], axis=1)
        scores = new_scores
        pad_profile = jnp.maximum(
            pad_profile[flat_beam], (token.reshape(-1) == EOS).astype(jnp.int32))

    return prefixes, scores.reshape(B, BEAM)


# speedup vs baseline: 1.0873x; 1.0873x over previous
import jax
import jax.numpy as jnp
from jax.experimental import pallas as pl
from jax.experimental.pallas import tpu as pltpu

V = 100000
D = 512
B = 16
BEAM = 4
STEPS = 6
PAD, BOS, EOS = 0, 1, 2
LEN_PEN = 0.6
NEG_INF = -1e9

VC = 10000
NC = V // VC
LANES = 128
NROWS = 64
NSUB = LANES // NROWS
SUB = VC // NSUB


def _topk_merge(sv, si, out_v, out_i):
    for t in range(BEAM):
        mv = jnp.max(sv, axis=0)
        hit = sv == mv[None, :]
        cidx = jnp.min(jnp.where(hit, si, V), axis=0)
        out_v[t, :] = mv
        out_i[t, :] = cidx
        if t + 1 < BEAM:
            sv = jnp.where(hit & (si == cidx[None, :]), NEG_INF, sv)


def _chunk_kernel(iota_ref, hT_ref, emb_ref, lse_ref, v_ref, i_ref,
                  acc_m, acc_s, acc_v, acc_i):
    j = pl.program_id(0)
    row = iota_ref[...]
    lane = jax.lax.broadcasted_iota(jnp.int32, (1, LANES), 1)
    sub_off = jnp.where(lane >= NROWS, SUB, 0)[0]

    tiles = [
        jax.lax.dot_general(
            emb_ref[s * SUB:(s + 1) * SUB, :], hT_ref[...],
            dimension_numbers=(((1,), (0,)), ((), ())),
            preferred_element_type=jnp.float32)
        for s in range(NSUB)
    ]
    logits = jnp.concatenate(tiles, axis=1)

    m = jnp.max(logits, axis=0)
    acc_m[pl.ds(j, 1), :] = m[None, :]
    acc_s[pl.ds(j, 1), :] = jnp.sum(jnp.exp(logits - m[None, :]),
                                    axis=0)[None, :]

    vals = logits
    cv, ci = [], []
    mt = m
    for t in range(BEAM):
        if t:
            mt = jnp.max(vals, axis=0)
        ki = jnp.where(vals == mt[None, :], row, SUB)
        at = jnp.min(ki, axis=0)
        cv.append(mt)
        ci.append(j * VC + at + sub_off)
        if t + 1 < BEAM:
            vals = jnp.where(ki == at[None, :], NEG_INF, vals)
    acc_v[pl.ds(j * BEAM, BEAM), :] = jnp.stack(cv, axis=0)
    acc_i[pl.ds(j * BEAM, BEAM), :] = jnp.stack(ci, axis=0)

    @pl.when(j == NC - 1)
    def _emit():
        m_all = acc_m[...]
        s_all = acc_s[...]
        mx = jnp.max(m_all, axis=0)
        sx = jnp.sum(s_all * jnp.exp(m_all - mx[None, :]), axis=0)
        m_lo, m_hi = mx[:NROWS], mx[NROWS:]
        s_lo, s_hi = sx[:NROWS], sx[NROWS:]
        mm = jnp.maximum(m_lo, m_hi)
        ss = s_lo * jnp.exp(m_lo - mm) + s_hi * jnp.exp(m_hi - mm)
        lse_ref[0, :] = mm + jnp.log(ss)
        v_all = acc_v[...]
        i_all = acc_i[...]
        sv = jnp.concatenate([v_all[:, :NROWS], v_all[:, NROWS:]], axis=0)
        si = jnp.concatenate([i_all[:, :NROWS], i_all[:, NROWS:]], axis=0)
        _topk_merge(sv, si, v_ref, i_ref)


def _vocab_topk(hT, emb_table):
    iota = jnp.broadcast_to(
        jnp.arange(SUB, dtype=jnp.int32)[:, None], (SUB, LANES))
    lse, top_v, top_i = pl.pallas_call(
        _chunk_kernel,
        grid=(NC,),
        in_specs=[
            pl.BlockSpec((SUB, LANES), lambda j: (0, 0)),
            pl.BlockSpec((D, NROWS), lambda j: (0, 0)),
            pl.BlockSpec((VC, D), lambda j: (j, 0)),
        ],
        out_specs=[
            pl.BlockSpec((1, NROWS), lambda j: (0, 0)),
            pl.BlockSpec((BEAM, NROWS), lambda j: (0, 0)),
            pl.BlockSpec((BEAM, NROWS), lambda j: (0, 0)),
        ],
        out_shape=[
            jax.ShapeDtypeStruct((1, NROWS), jnp.float32),
            jax.ShapeDtypeStruct((BEAM, NROWS), jnp.float32),
            jax.ShapeDtypeStruct((BEAM, NROWS), jnp.int32),
        ],
        scratch_shapes=[
            pltpu.VMEM((NC, LANES), jnp.float32),
            pltpu.VMEM((NC, LANES), jnp.float32),
            pltpu.VMEM((NC * BEAM, LANES), jnp.float32),
            pltpu.VMEM((NC * BEAM, LANES), jnp.int32),
        ],
    )(iota, hT, emb_table)
    return top_v.T, top_i.T, lse[0]


def kernel(decoder_input_ids, encoder_hidden_states, encoder_input_mask,
           emb_table, pos_emb, W_dec, W_enc):
    mask = encoder_input_mask
    enc_ctx = (encoder_hidden_states * mask[:, :, None]).sum(axis=1) / \
        jnp.maximum(mask.sum(axis=1, keepdims=True), 1.0)

    tok0 = decoder_input_ids[:, 0]
    h = jnp.take(emb_table, tok0, axis=0) + pos_emb[0][None, :]
    h = jnp.tanh(h @ W_dec + enc_ctx @ W_enc)
    h = jnp.tile(h, (NROWS // B, 1))
    top_v, top_i, lse = _vocab_topk(h.T, emb_table)
    top_v, top_i, lse = top_v[:B], top_i[:B], lse[:B]
    scores = (top_v - lse[:, None]).reshape(-1, 1)
    prefixes = jnp.concatenate(
        [jnp.repeat(decoder_input_ids, BEAM, axis=0),
         top_i.reshape(-1, 1).astype(jnp.int32)], axis=1)
    ctx_rep = jnp.repeat(enc_ctx, BEAM, axis=0)
    ctx_enc = ctx_rep @ W_enc
    pad_profile = jnp.zeros((B * BEAM,), dtype=jnp.int32)

    for i in range(1, STEPS + 1):
        tok = prefixes[:, -1]
        h = jnp.take(emb_table, tok, axis=0) + pos_emb[i][None, :]
        h = jnp.tanh(h @ W_dec + ctx_enc)
        top_v, top_i, lse = _vocab_topk(h.T, emb_table)
        lp_cand = top_v - lse[:, None]
        finished = pad_profile > 0
        pad_lp = jnp.where(jnp.arange(BEAM) == 0, 0.0, NEG_INF)
        lp_cand = jnp.where(finished[:, None], pad_lp[None, :], lp_cand)
        tok_cand = jnp.where(finished[:, None], PAD, top_i)

        total = scores + lp_cand
        length = prefixes.shape[1]
        penalty = ((5.0 + length) / 6.0) ** LEN_PEN
        cand = (total / penalty).reshape(B, BEAM * BEAM)
        _, tpos = jax.lax.top_k(cand, BEAM)
        beam_idx = tpos // BEAM
        token = jnp.take_along_axis(
            tok_cand.reshape(B, BEAM * BEAM), tpos, axis=1).astype(jnp.int32)
        new_scores = jnp.take_along_axis(
            total.reshape(B, BEAM * BEAM), tpos, axis=1).reshape(-1, 1)
        flat_beam = (beam_idx + jnp.arange(B)[:, None] * BEAM).reshape(-1)
        prefixes = jnp.concatenate(
            [prefixes[flat_beam], token.reshape(-1, 1)], axis=1)
        scores = new_scores
        pad_profile = jnp.maximum(
            pad_profile[flat_beam], (token.reshape(-1) == EOS).astype(jnp.int32))

    return prefixes, scores.reshape(B, BEAM)


# speedup vs baseline: 1.0885x; 1.0011x over previous
import jax
import jax.numpy as jnp
from jax.experimental import pallas as pl
from jax.experimental.pallas import tpu as pltpu

V = 100000
D = 512
B = 16
BEAM = 4
STEPS = 6
PAD, BOS, EOS = 0, 1, 2
LEN_PEN = 0.6
NEG_INF = -1e9

VC = 10000
NC = V // VC
LANES = 128
NROWS = 64
NSUB = LANES // NROWS
SUB = VC // NSUB


def _topk_merge(sv, si, out_v, out_i):
    for t in range(BEAM):
        mv = jnp.max(sv, axis=0)
        hit = sv == mv[None, :]
        cidx = jnp.min(jnp.where(hit, si, V), axis=0)
        out_v[t, :] = mv
        out_i[t, :] = cidx
        if t + 1 < BEAM:
            sv = jnp.where(hit & (si == cidx[None, :]), NEG_INF, sv)


def _chunk_kernel(iota_ref, hT_ref, emb_ref, lse_ref, v_ref, i_ref,
                  acc_m, acc_s, acc_v, acc_i):
    j = pl.program_id(0)
    row = iota_ref[...]
    lane = jax.lax.broadcasted_iota(jnp.int32, (1, LANES), 1)
    sub_off = jnp.where(lane >= NROWS, SUB, 0)[0]

    tiles = [
        jax.lax.dot_general(
            emb_ref[s * SUB:(s + 1) * SUB, :], hT_ref[...],
            dimension_numbers=(((1,), (0,)), ((), ())),
            preferred_element_type=jnp.float32)
        for s in range(NSUB)
    ]
    logits = jnp.concatenate(tiles, axis=1)

    m = jnp.max(logits, axis=0)
    acc_m[pl.ds(j, 1), :] = m[None, :]
    acc_s[pl.ds(j, 1), :] = jnp.sum(jnp.exp(logits - m[None, :]),
                                    axis=0)[None, :]

    vals = logits
    cv, ci = [], []
    mt = m
    for t in range(BEAM):
        if t:
            mt = jnp.max(vals, axis=0)
        at = jnp.min(jnp.where(vals == mt[None, :], row, SUB), axis=0)
        cv.append(mt)
        ci.append(j * VC + at + sub_off)
        if t + 1 < BEAM:
            vals = jnp.where(row == at[None, :], NEG_INF, vals)
    acc_v[pl.ds(j * BEAM, BEAM), :] = jnp.stack(cv, axis=0)
    acc_i[pl.ds(j * BEAM, BEAM), :] = jnp.stack(ci, axis=0)

    @pl.when(j == NC - 1)
    def _emit():
        m_all = acc_m[...]
        s_all = acc_s[...]
        mx = jnp.max(m_all, axis=0)
        sx = jnp.sum(s_all * jnp.exp(m_all - mx[None, :]), axis=0)
        m_lo, m_hi = mx[:NROWS], mx[NROWS:]
        s_lo, s_hi = sx[:NROWS], sx[NROWS:]
        mm = jnp.maximum(m_lo, m_hi)
        ss = s_lo * jnp.exp(m_lo - mm) + s_hi * jnp.exp(m_hi - mm)
        lse_ref[0, :] = mm + jnp.log(ss)
        v_all = acc_v[...]
        i_all = acc_i[...]
        sv = jnp.concatenate([v_all[:, :NROWS], v_all[:, NROWS:]], axis=0)
        si = jnp.concatenate([i_all[:, :NROWS], i_all[:, NROWS:]], axis=0)
        _topk_merge(sv, si, v_ref, i_ref)


def _vocab_topk(hT, emb_table):
    iota = jnp.broadcast_to(
        jnp.arange(SUB, dtype=jnp.int32)[:, None], (SUB, LANES))
    lse, top_v, top_i = pl.pallas_call(
        _chunk_kernel,
        grid=(NC,),
        in_specs=[
            pl.BlockSpec((SUB, LANES), lambda j: (0, 0)),
            pl.BlockSpec((D, NROWS), lambda j: (0, 0)),
            pl.BlockSpec((VC, D), lambda j: (j, 0)),
        ],
        out_specs=[
            pl.BlockSpec((1, NROWS), lambda j: (0, 0)),
            pl.BlockSpec((BEAM, NROWS), lambda j: (0, 0)),
            pl.BlockSpec((BEAM, NROWS), lambda j: (0, 0)),
        ],
        out_shape=[
            jax.ShapeDtypeStruct((1, NROWS), jnp.float32),
            jax.ShapeDtypeStruct((BEAM, NROWS), jnp.float32),
            jax.ShapeDtypeStruct((BEAM, NROWS), jnp.int32),
        ],
        scratch_shapes=[
            pltpu.VMEM((NC, LANES), jnp.float32),
            pltpu.VMEM((NC, LANES), jnp.float32),
            pltpu.VMEM((NC * BEAM, LANES), jnp.float32),
            pltpu.VMEM((NC * BEAM, LANES), jnp.int32),
        ],
    )(iota, hT, emb_table)
    return top_v.T, top_i.T, lse[0]


def kernel(decoder_input_ids, encoder_hidden_states, encoder_input_mask,
           emb_table, pos_emb, W_dec, W_enc):
    mask = encoder_input_mask
    enc_ctx = (encoder_hidden_states * mask[:, :, None]).sum(axis=1) / \
        jnp.maximum(mask.sum(axis=1, keepdims=True), 1.0)

    tok0 = decoder_input_ids[:, 0]
    h = jnp.take(emb_table, tok0, axis=0) + pos_emb[0][None, :]
    h = jnp.tanh(h @ W_dec + enc_ctx @ W_enc)
    h = jnp.tile(h, (NROWS // B, 1))
    top_v, top_i, lse = _vocab_topk(h.T, emb_table)
    top_v, top_i, lse = top_v[:B], top_i[:B], lse[:B]
    scores = (top_v - lse[:, None]).reshape(-1, 1)
    prefixes = jnp.concatenate(
        [jnp.repeat(decoder_input_ids, BEAM, axis=0),
         top_i.reshape(-1, 1).astype(jnp.int32)], axis=1)
    ctx_rep = jnp.repeat(enc_ctx, BEAM, axis=0)
    ctx_enc = ctx_rep @ W_enc
    pad_profile = jnp.zeros((B * BEAM,), dtype=jnp.int32)

    for i in range(1, STEPS + 1):
        tok = prefixes[:, -1]
        h = jnp.take(emb_table, tok, axis=0) + pos_emb[i][None, :]
        h = jnp.tanh(h @ W_dec + ctx_enc)
        top_v, top_i, lse = _vocab_topk(h.T, emb_table)
        lp_cand = top_v - lse[:, None]
        finished = pad_profile > 0
        pad_lp = jnp.where(jnp.arange(BEAM) == 0, 0.0, NEG_INF)
        lp_cand = jnp.where(finished[:, None], pad_lp[None, :], lp_cand)
        tok_cand = jnp.where(finished[:, None], PAD, top_i)

        total = scores + lp_cand
        length = prefixes.shape[1]
        penalty = ((5.0 + length) / 6.0) ** LEN_PEN
        cand = (total / penalty).reshape(B, BEAM * BEAM)
        _, tpos = jax.lax.top_k(cand, BEAM)
        beam_idx = tpos // BEAM
        token = jnp.take_along_axis(
            tok_cand.reshape(B, BEAM * BEAM), tpos, axis=1).astype(jnp.int32)
        new_scores = jnp.take_along_axis(
            total.reshape(B, BEAM * BEAM), tpos, axis=1).reshape(-1, 1)
        flat_beam = (beam_idx + jnp.arange(B)[:, None] * BEAM).reshape(-1)
        prefixes = jnp.concatenate(
            [prefixes[flat_beam], token.reshape(-1, 1)], axis=1)
        scores = new_scores
        pad_profile = jnp.maximum(
            pad_profile[flat_beam], (token.reshape(-1) == EOS).astype(jnp.int32))

    return prefixes, scores.reshape(B, BEAM)
